# baseline (speedup 1.0000x reference)
import functools

import jax
import jax.numpy as jnp
from jax import lax
from jax.experimental import pallas as pl
from jax.experimental.pallas import tpu as pltpu
from jax.experimental.pallas import tpu_sc as plsc

_CLASS_NUM = 100000
_DIM = 128
_BATCH = 16384

_NC = 2
_NS = 16
_NW = _NC * _NS
_BPW = _BATCH // _NW
_CHUNK = 128
_NCHUNK = _BPW // _CHUNK
_LANES = 16
_M = 0.9
_ROWS_PER_W = 3128
_ROWS_LAST = _CLASS_NUM - (_NW - 1) * _ROWS_PER_W


def _copy_gather_ema(keys, index, que):
  mesh = plsc.VectorSubcoreMesh(core_axis_name="c", subcore_axis_name="s")

  @functools.partial(
      pl.kernel,
      mesh=mesh,
      out_type=(
          jax.ShapeDtypeStruct((_CLASS_NUM, _DIM), jnp.float32),
          jax.ShapeDtypeStruct((_BATCH, _DIM), jnp.float32),
      ),
      scratch_types=[
          pltpu.VMEM((_NCHUNK, _CHUNK), jnp.int32),
          pltpu.VMEM((_BPW, _DIM), jnp.float32),
          pltpu.VMEM((_DIM,), jnp.float32),
          [pltpu.SemaphoreType.DMA] * _NCHUNK,
          pltpu.SemaphoreType.DMA,
          pltpu.SemaphoreType.DMA,
      ],
  )
  def k(keys_hbm, idx_hbm, que_hbm, out_hbm, rows_hbm, idx_v, rows_v, keys_v,
        gsems, wsem, csem):
    wid = lax.axis_index("s") * _NC + lax.axis_index("c")
    base = wid * _BPW
    rbase = pl.multiple_of(wid * _ROWS_PER_W, 8)
    is_last = wid == _NW - 1

    @pl.when(jnp.logical_not(is_last))
    def _():
      pltpu.async_copy(
          que_hbm.at[pl.ds(rbase, _ROWS_PER_W)],
          out_hbm.at[pl.ds(rbase, _ROWS_PER_W)],
          csem,
      )

    @pl.when(is_last)
    def _():
      pltpu.async_copy(
          que_hbm.at[pl.ds(rbase, _ROWS_LAST)],
          out_hbm.at[pl.ds(rbase, _ROWS_LAST)],
          csem,
      )

    pltpu.sync_copy(keys_hbm, keys_v)
    for j in range(_NCHUNK):
      pltpu.sync_copy(
          idx_hbm.at[pl.ds(base + j * _CHUNK, _CHUNK)], idx_v.at[j]
      )

    gathers = [
        pltpu.async_copy(
            que_hbm.at[idx_v.at[j]],
            rows_v.at[pl.ds(j * _CHUNK, _CHUNK)],
            gsems[j],
        )
        for j in range(_NCHUNK)
    ]

    kc = [keys_v[pl.ds(c * _LANES, _LANES)] * (1.0 - _M)
          for c in range(_DIM // _LANES)]

    writes = []
    for j in range(_NCHUNK):
      gathers[j].wait()
      lo = j * _CHUNK

      def row_body(r):
        for c in range(_DIM // _LANES):
          sl = pl.ds(c * _LANES, _LANES)
          rows_v[r, sl] = rows_v[r, sl] * _M + kc[c]

      plsc.parallel_loop(lo, lo + _CHUNK, unroll=4)(row_body)

      writes.append(
          pltpu.async_copy(
              rows_v.at[pl.ds(lo, _CHUNK)],
              rows_hbm.at[pl.ds(base + lo, _CHUNK)],
              wsem,
          )
      )
    for w in writes:
      w.wait()

    @pl.when(jnp.logical_not(is_last))
    def _():
      pltpu.make_async_copy(
          que_hbm.at[pl.ds(rbase, _ROWS_PER_W)],
          out_hbm.at[pl.ds(rbase, _ROWS_PER_W)],
          csem,
      ).wait()

    @pl.when(is_last)
    def _():
      pltpu.make_async_copy(
          que_hbm.at[pl.ds(rbase, _ROWS_LAST)],
          out_hbm.at[pl.ds(rbase, _ROWS_LAST)],
          csem,
      ).wait()

  return k(keys, index, que)


def _scatter(index, new_rows, out_ref):
  mesh = plsc.VectorSubcoreMesh(core_axis_name="c", subcore_axis_name="s")

  @functools.partial(
      pl.kernel,
      mesh=mesh,
      out_type=(),
      scratch_types=[
          pltpu.VMEM((_NCHUNK, _CHUNK), jnp.int32),
          pltpu.VMEM((_BPW, _DIM), jnp.float32),
          [pltpu.SemaphoreType.DMA] * _NCHUNK,
          pltpu.SemaphoreType.DMA,
      ],
  )
  def k(idx_hbm, rows_hbm, out_hbm, idx_v, rows_v, ssems, scsem):
    wid = lax.axis_index("s") * _NC + lax.axis_index("c")
    base = wid * _BPW

    stages = [
        pltpu.async_copy(
            rows_hbm.at[pl.ds(base + j * _CHUNK, _CHUNK)],
            rows_v.at[pl.ds(j * _CHUNK, _CHUNK)],
            ssems[j],
        )
        for j in range(_NCHUNK)
    ]
    for j in range(_NCHUNK):
      pltpu.sync_copy(
          idx_hbm.at[pl.ds(base + j * _CHUNK, _CHUNK)], idx_v.at[j]
      )

    scatters = []
    for j in range(_NCHUNK):
      stages[j].wait()
      scatters.append(
          pltpu.async_copy(
              rows_v.at[pl.ds(j * _CHUNK, _CHUNK)],
              out_hbm.at[idx_v.at[j]],
              scsem,
          )
      )
    for s in scatters:
      s.wait()

  k(index, new_rows, out_ref)


def kernel(keys, index, que):
  idx = index.astype(jnp.int32)
  table, new_rows = _copy_gather_ema(keys, idx, que)
  out_ref = jax.new_ref(table)
  _scatter(idx, new_rows, out_ref)
  return jax.freeze(out_ref)


# speedup vs baseline: 27.8190x; 27.8190x over previous
import functools

import jax
import jax.numpy as jnp
from jax import lax
from jax.experimental import pallas as pl
from jax.experimental.pallas import tpu as pltpu
from jax.experimental.pallas import tpu_sc as plsc

_CLASS_NUM = 100000
_DIM = 128
_BATCH = 16384

_NC = 2
_NS = 16
_NW = _NC * _NS
_BPW = _BATCH // _NW
_CHUNK = 128
_NCHUNK = _BPW // _CHUNK
_LANES = 16
_M = 0.9


def _sc_update(keys, index, que, out_ref):
  mesh = plsc.VectorSubcoreMesh(core_axis_name="c", subcore_axis_name="s")

  @functools.partial(
      pl.kernel,
      mesh=mesh,
      out_type=(),
      scratch_types=[
          pltpu.VMEM((_NCHUNK, _CHUNK), jnp.int32),
          pltpu.VMEM((_BPW, _DIM), jnp.float32),
          pltpu.VMEM((_DIM,), jnp.float32),
          [pltpu.SemaphoreType.DMA] * _NCHUNK,
          [pltpu.SemaphoreType.DMA] * _NCHUNK,
          pltpu.SemaphoreType.DMA,
          pltpu.SemaphoreType.DMA,
      ],
  )
  def k(keys_hbm, idx_hbm, que_hbm, out_hbm, idx_v, rows_v, keys_v, gsems,
        isems, ksem, ssem):
    wid = lax.axis_index("s") * _NC + lax.axis_index("c")
    base = wid * _BPW

    kcopy = pltpu.async_copy(keys_hbm, keys_v, ksem)
    icopies = [
        pltpu.async_copy(
            idx_hbm.at[pl.ds(base + j * _CHUNK, _CHUNK)], idx_v.at[j],
            isems[j],
        )
        for j in range(_NCHUNK)
    ]

    gathers = []
    for j in range(_NCHUNK):
      icopies[j].wait()
      gathers.append(
          pltpu.async_copy(
              que_hbm.at[idx_v.at[j]],
              rows_v.at[pl.ds(j * _CHUNK, _CHUNK)],
              gsems[j],
          )
      )

    kcopy.wait()
    kc = [keys_v[pl.ds(c * _LANES, _LANES)] * (1.0 - _M)
          for c in range(_DIM // _LANES)]

    scatters = []
    for j in range(_NCHUNK):
      gathers[j].wait()
      lo = j * _CHUNK

      def row_body(r):
        for c in range(_DIM // _LANES):
          sl = pl.ds(c * _LANES, _LANES)
          rows_v[r, sl] = rows_v[r, sl] * _M + kc[c]

      plsc.parallel_loop(lo, lo + _CHUNK, unroll=8)(row_body)

      scatters.append(
          pltpu.async_copy(
              rows_v.at[pl.ds(lo, _CHUNK)],
              out_hbm.at[idx_v.at[j]],
              ssem,
          )
      )
    for s in scatters:
      s.wait()

  k(keys, index, que, out_ref)


def kernel(keys, index, que):
  out_ref = jax.new_ref(que)
  _sc_update(keys, index.astype(jnp.int32), que, out_ref)
  return jax.freeze(out_ref)
